# baseline (speedup 1.0000x reference)
import functools
import jax
import jax.numpy as jnp
from jax import lax
from jax.experimental import pallas as pl
from jax.experimental.pallas import tpu as pltpu
from jax.experimental.pallas import tpu_sc as plsc

N = 10000
E = 320000
NP = 10240
NC, NS = 2, 16
NW = NC * NS
EPW = E // NW
C = 128
NCHUNK = 80
EPWP = NCHUNK * C
RPT = NP // NS
DW = 8
DH = 64
D3 = 40
KH = 78

_mesh = plsc.VectorSubcoreMesh(
    core_axis_name="c", subcore_axis_name="s", num_cores=NC, num_subcores=NS)
_untiled = pltpu.CompilerParams(use_tc_tiling_on_sc=False)


@functools.partial(
    pl.kernel,
    out_type=jax.ShapeDtypeStruct((NC, NP, DW), jnp.float32),
    mesh=_mesh,
    compiler_params=_untiled,
    scratch_types=[
        pltpu.VMEM((NCHUNK, C), jnp.int32),
        pltpu.VMEM((C, DW), jnp.float32),
        pltpu.VMEM_SHARED((NP, DW), jnp.float32),
    ],
)
def _deg_kernel(dst3_hbm, zs_hbm, out_hbm, dstv, ones, acc):
    c = lax.axis_index("c")
    s = lax.axis_index("s")
    wid = s * NC + c

    def fill(i, carry):
        ones[i, :] = jnp.ones((DW,), jnp.float32)
        return carry
    lax.fori_loop(0, C, fill, 0)

    pltpu.sync_copy(zs_hbm.at[pl.ds(s * RPT, RPT)], acc.at[pl.ds(s * RPT, RPT)])
    pltpu.sync_copy(dst3_hbm.at[wid], dstv)
    plsc.subcore_barrier()

    def body(i, carry):
        pltpu.sync_copy(ones, acc.at[dstv.at[i]], add=True)
        return carry
    lax.fori_loop(0, NCHUNK, body, 0)

    plsc.subcore_barrier()
    pltpu.sync_copy(acc.at[pl.ds(s * RPT, RPT)],
                    out_hbm.at[c, pl.ds(s * RPT, RPT)])


@functools.partial(
    pl.kernel,
    out_type=jax.ShapeDtypeStruct((NC, NP, DH), jnp.float32),
    mesh=_mesh,
    compiler_params=_untiled,
    scratch_types=[
        pltpu.VMEM((EPWP,), jnp.int32),
        pltpu.VMEM((NCHUNK, C), jnp.int32),
        pltpu.VMEM((C, DH), jnp.float32),
        pltpu.VMEM((C, DH), jnp.float32),
        pltpu.VMEM_SHARED((NP, DH), jnp.float32),
        pltpu.VMEM_SHARED((NP, DH), jnp.float32),
        pltpu.SemaphoreType.DMA,
        pltpu.SemaphoreType.DMA,
    ],
)
def _spmm_split(h2_hbm, src2_hbm, dst3_hbm, zs_hbm, out_hbm,
                srcf, dstv, rows_a, rows_b, h_sh, acc, sem_a, sem_b):
    c = lax.axis_index("c")
    s = lax.axis_index("s")

    if KH < NCHUNK:
        pltpu.sync_copy(h2_hbm.at[c, pl.ds(s * RPT, RPT)],
                        h_sh.at[pl.ds(s * RPT, RPT)])
    pltpu.sync_copy(zs_hbm.at[pl.ds(s * RPT, RPT)],
                    acc.at[pl.ds(s * RPT, RPT)])
    plsc.subcore_barrier()

    hbase = h2_hbm.at[c]

    def wt(buf, sem):
        pltpu.make_async_copy(h_sh.at[srcf.at[pl.ds(0, C)]], buf, sem).wait()

    def run_phase(lo, hi, table):
        def gat(j, buf, sem):
            pltpu.async_copy(table.at[srcf.at[pl.ds(j * C, C)]], buf, sem)

        gat(lo, rows_a, sem_a)

        def body(k, carry):
            a = lo + 2 * k
            b = a + 1
            gat(b, rows_b, sem_b)
            wt(rows_a, sem_a)
            pltpu.sync_copy(rows_a, acc.at[dstv.at[a]], add=True)

            @pl.when(b + 1 < hi)
            def _():
                gat(b + 1, rows_a, sem_a)
            wt(rows_b, sem_b)
            pltpu.sync_copy(rows_b, acc.at[dstv.at[b]], add=True)
            return carry
        lax.fori_loop(0, (hi - lo) // 2, body, 0)

    for p in range(2):
        w = s * 2 + p
        pltpu.sync_copy(src2_hbm.at[w], srcf)
        pltpu.sync_copy(dst3_hbm.at[w], dstv)
        if KH > 0:
            run_phase(0, KH, hbase)
        if KH < NCHUNK:
            run_phase(KH, NCHUNK, h_sh)

    plsc.subcore_barrier()
    pltpu.sync_copy(acc.at[pl.ds(s * RPT, RPT)],
                    out_hbm.at[c, pl.ds(s * RPT, RPT)])


@functools.partial(
    pl.kernel,
    out_type=jax.ShapeDtypeStruct((NC, NP, D3), jnp.float32),
    mesh=_mesh,
    compiler_params=_untiled,
    scratch_types=[
        pltpu.VMEM((EPWP,), jnp.int32),
        pltpu.VMEM((NCHUNK, C), jnp.int32),
        pltpu.VMEM((C, D3), jnp.float32),
        pltpu.VMEM((C, D3), jnp.float32),
        pltpu.VMEM_SHARED((NP, D3), jnp.float32),
        pltpu.VMEM_SHARED((NP, D3), jnp.float32),
        pltpu.SemaphoreType.DMA,
        pltpu.SemaphoreType.DMA,
    ],
)
def _spmm40(h_hbm, src2_hbm, dst3_hbm, zs_hbm, out_hbm,
            srcf, dstv, rows_a, rows_b, h_sh, acc, sem_a, sem_b):
    c = lax.axis_index("c")
    s = lax.axis_index("s")
    wid = s * NC + c

    pltpu.sync_copy(h_hbm.at[pl.ds(s * RPT, RPT)],
                    h_sh.at[pl.ds(s * RPT, RPT)])
    pltpu.sync_copy(zs_hbm.at[pl.ds(s * RPT, RPT)],
                    acc.at[pl.ds(s * RPT, RPT)])
    pltpu.sync_copy(src2_hbm.at[wid], srcf)
    pltpu.sync_copy(dst3_hbm.at[wid], dstv)
    plsc.subcore_barrier()

    def gat(j, buf, sem):
        pltpu.async_copy(h_sh.at[srcf.at[pl.ds(j * C, C)]], buf, sem)

    def wt(buf, sem):
        pltpu.make_async_copy(h_sh.at[srcf.at[pl.ds(0, C)]], buf, sem).wait()

    gat(0, rows_a, sem_a)

    def body(k, carry):
        a = 2 * k
        b = a + 1
        gat(b, rows_b, sem_b)
        wt(rows_a, sem_a)
        pltpu.sync_copy(rows_a, acc.at[dstv.at[a]], add=True)

        @pl.when(b + 1 < NCHUNK)
        def _():
            gat(b + 1, rows_a, sem_a)
        wt(rows_b, sem_b)
        pltpu.sync_copy(rows_b, acc.at[dstv.at[b]], add=True)
        return carry
    lax.fori_loop(0, NCHUNK // 2, body, 0)

    plsc.subcore_barrier()
    pltpu.sync_copy(acc.at[pl.ds(s * RPT, RPT)],
                    out_hbm.at[c, pl.ds(s * RPT, RPT)])


def _dense1(xp, W1, deg2):
    def body(x_ref, w_ref, d_ref, h_ref, dinv_ref):
        deg = d_ref[0][:, 0:1] + d_ref[1][:, 0:1] + 1.0
        dinv = lax.rsqrt(jnp.maximum(deg, 1.0))
        h = jnp.dot(x_ref[...], w_ref[...], preferred_element_type=jnp.float32)
        hp = h * dinv
        h_ref[0] = hp[:, :DH]
        h_ref[1] = hp[:, DH:]
        dinv_ref[...] = dinv
    return pl.pallas_call(
        body,
        out_shape=[jax.ShapeDtypeStruct((NC, NP, DH), jnp.float32),
                   jax.ShapeDtypeStruct((NP, 1), jnp.float32)],
    )(xp, W1, deg2)


def _dense_mid(S, hp2, dinv, b, g, be, W, DO):
    def body(S_ref, hp_ref, dinv_ref, b_ref, g_ref, be_ref, w_ref, o_ref):
        a = jnp.concatenate(
            [S_ref[0] + hp_ref[0], S_ref[1] + hp_ref[1]], axis=1)
        a = a * dinv_ref[...] + b_ref[...]
        aN = a[:N]
        m = jnp.mean(aN, axis=0, keepdims=True)
        d = aN - m
        v = jnp.mean(d * d, axis=0, keepdims=True)
        y = (a - m) * lax.rsqrt(v + 1e-5) * g_ref[...] + be_ref[...]
        y = jnp.maximum(y, 0.0)
        h = jnp.dot(y, w_ref[...], preferred_element_type=jnp.float32)
        hp = h * dinv_ref[...]
        if DO == 2 * DH:
            o_ref[0] = hp[:, :DH]
            o_ref[1] = hp[:, DH:]
        else:
            o_ref[...] = hp
    out_shape = (jax.ShapeDtypeStruct((NC, NP, DH), jnp.float32)
                 if DO == 2 * DH else
                 jax.ShapeDtypeStruct((NP, DO), jnp.float32))
    return pl.pallas_call(
        body,
        out_shape=out_shape,
    )(S, hp2, dinv, b, g, be, W)


def _dense_final(S, hp, dinv, b):
    def body(S_ref, hp_ref, dinv_ref, b_ref, o_ref):
        a = (S_ref[0] + S_ref[1] + hp_ref[...]) * dinv_ref[...] + b_ref[...]
        aN = a[:N, :D3]
        z = aN - jnp.max(aN, axis=1, keepdims=True)
        lse = jnp.log(jnp.sum(jnp.exp(z), axis=1, keepdims=True))
        o_ref[...] = z - lse
    return pl.pallas_call(
        body,
        out_shape=jax.ShapeDtypeStruct((N, D3), jnp.float32),
    )(S, hp, dinv, b)


def kernel(adj_t, x, W1, b1, g1, be1, W2, b2, g2, be2, W3, b3):
    f32 = jnp.float32
    pad = ((0, 0), (0, EPWP - EPW))
    src2 = jnp.pad(adj_t[0].astype(jnp.int32).reshape(NW, EPW), pad,
                   constant_values=NP - 1)
    dst3 = jnp.pad(adj_t[1].astype(jnp.int32).reshape(NW, EPW), pad,
                   constant_values=NP - 1).reshape(NW, NCHUNK, C)
    xp = jnp.pad(x.astype(f32), ((0, NP - N), (0, 0)))
    zs8 = jnp.zeros((NP, DW), f32)
    zs64 = jnp.zeros((NP, DH), f32)
    zs40 = jnp.zeros((NP, D3), f32)
    b1r = b1.reshape(1, 128)
    g1r = g1.reshape(1, 128)
    be1r = be1.reshape(1, 128)
    b2r = b2.reshape(1, 128)
    g2r = g2.reshape(1, 128)
    be2r = be2.reshape(1, 128)
    b3r = b3.reshape(1, D3)

    deg2 = _deg_kernel(dst3, zs8)
    h1p2, dinv = _dense1(xp, W1, deg2)
    S1 = _spmm_split(h1p2, src2, dst3, zs64)
    h2p2 = _dense_mid(S1, h1p2, dinv, b1r, g1r, be1r, W2, 128)
    S2 = _spmm_split(h2p2, src2, dst3, zs64)
    h3p = _dense_mid(S2, h2p2, dinv, b2r, g2r, be2r, W3, D3)
    S3 = _spmm40(h3p, src2, dst3, zs40)
    return _dense_final(S3, h3p, dinv, b3r)


# speedup vs baseline: 1.0015x; 1.0015x over previous
import functools
import jax
import jax.numpy as jnp
from jax import lax
from jax.experimental import pallas as pl
from jax.experimental.pallas import tpu as pltpu
from jax.experimental.pallas import tpu_sc as plsc

N = 10000
E = 320000
NP = 10240
NC, NS = 2, 16
NW = NC * NS
EPW = E // NW
C = 128
NCHUNK = 80
EPWP = NCHUNK * C
RPT = NP // NS
DW = 8
DH = 64
D3 = 40
KH = 76

_mesh = plsc.VectorSubcoreMesh(
    core_axis_name="c", subcore_axis_name="s", num_cores=NC, num_subcores=NS)
_untiled = pltpu.CompilerParams(use_tc_tiling_on_sc=False)


@functools.partial(
    pl.kernel,
    out_type=jax.ShapeDtypeStruct((NC, NP, DW), jnp.float32),
    mesh=_mesh,
    compiler_params=_untiled,
    scratch_types=[
        pltpu.VMEM((NCHUNK, C), jnp.int32),
        pltpu.VMEM((C, DW), jnp.float32),
        pltpu.VMEM_SHARED((NP, DW), jnp.float32),
    ],
)
def _deg_kernel(dst3_hbm, zs_hbm, out_hbm, dstv, ones, acc):
    c = lax.axis_index("c")
    s = lax.axis_index("s")
    wid = s * NC + c

    def fill(i, carry):
        ones[i, :] = jnp.ones((DW,), jnp.float32)
        return carry
    lax.fori_loop(0, C, fill, 0)

    pltpu.sync_copy(zs_hbm.at[pl.ds(s * RPT, RPT)], acc.at[pl.ds(s * RPT, RPT)])
    pltpu.sync_copy(dst3_hbm.at[wid], dstv)
    plsc.subcore_barrier()

    def body(i, carry):
        pltpu.sync_copy(ones, acc.at[dstv.at[i]], add=True)
        return carry
    lax.fori_loop(0, NCHUNK, body, 0)

    plsc.subcore_barrier()
    pltpu.sync_copy(acc.at[pl.ds(s * RPT, RPT)],
                    out_hbm.at[c, pl.ds(s * RPT, RPT)])


@functools.partial(
    pl.kernel,
    out_type=jax.ShapeDtypeStruct((NC, NP, DH), jnp.float32),
    mesh=_mesh,
    compiler_params=_untiled,
    scratch_types=[
        pltpu.VMEM((EPWP,), jnp.int32),
        pltpu.VMEM((NCHUNK, C), jnp.int32),
        pltpu.VMEM((C, DH), jnp.float32),
        pltpu.VMEM((C, DH), jnp.float32),
        pltpu.VMEM_SHARED((NP, DH), jnp.float32),
        pltpu.VMEM_SHARED((NP, DH), jnp.float32),
        pltpu.SemaphoreType.DMA,
        pltpu.SemaphoreType.DMA,
    ],
)
def _spmm_split(h2_hbm, src2_hbm, dst3_hbm, zs_hbm, out_hbm,
                srcf, dstv, rows_a, rows_b, h_sh, acc, sem_a, sem_b):
    c = lax.axis_index("c")
    s = lax.axis_index("s")

    if KH < NCHUNK:
        pltpu.sync_copy(h2_hbm.at[c, pl.ds(s * RPT, RPT)],
                        h_sh.at[pl.ds(s * RPT, RPT)])
    pltpu.sync_copy(zs_hbm.at[pl.ds(s * RPT, RPT)],
                    acc.at[pl.ds(s * RPT, RPT)])
    plsc.subcore_barrier()

    hbase = h2_hbm.at[c]

    def wt(buf, sem):
        pltpu.make_async_copy(h_sh.at[srcf.at[pl.ds(0, C)]], buf, sem).wait()

    def run_phase(lo, hi, table):
        def gat(j, buf, sem):
            pltpu.async_copy(table.at[srcf.at[pl.ds(j * C, C)]], buf, sem)

        gat(lo, rows_a, sem_a)

        def body(k, carry):
            a = lo + 2 * k
            b = a + 1
            gat(b, rows_b, sem_b)
            wt(rows_a, sem_a)
            pltpu.sync_copy(rows_a, acc.at[dstv.at[a]], add=True)

            @pl.when(b + 1 < hi)
            def _():
                gat(b + 1, rows_a, sem_a)
            wt(rows_b, sem_b)
            pltpu.sync_copy(rows_b, acc.at[dstv.at[b]], add=True)
            return carry
        lax.fori_loop(0, (hi - lo) // 2, body, 0)

    for p in range(2):
        w = s * 2 + p
        pltpu.sync_copy(src2_hbm.at[w], srcf)
        pltpu.sync_copy(dst3_hbm.at[w], dstv)
        if KH > 0:
            run_phase(0, KH, hbase)
        if KH < NCHUNK:
            run_phase(KH, NCHUNK, h_sh)

    plsc.subcore_barrier()
    pltpu.sync_copy(acc.at[pl.ds(s * RPT, RPT)],
                    out_hbm.at[c, pl.ds(s * RPT, RPT)])


@functools.partial(
    pl.kernel,
    out_type=jax.ShapeDtypeStruct((NC, NP, D3), jnp.float32),
    mesh=_mesh,
    compiler_params=_untiled,
    scratch_types=[
        pltpu.VMEM((EPWP,), jnp.int32),
        pltpu.VMEM((NCHUNK, C), jnp.int32),
        pltpu.VMEM((C, D3), jnp.float32),
        pltpu.VMEM((C, D3), jnp.float32),
        pltpu.VMEM_SHARED((NP, D3), jnp.float32),
        pltpu.VMEM_SHARED((NP, D3), jnp.float32),
        pltpu.SemaphoreType.DMA,
        pltpu.SemaphoreType.DMA,
    ],
)
def _spmm40(h_hbm, src2_hbm, dst3_hbm, zs_hbm, out_hbm,
            srcf, dstv, rows_a, rows_b, h_sh, acc, sem_a, sem_b):
    c = lax.axis_index("c")
    s = lax.axis_index("s")
    wid = s * NC + c

    pltpu.sync_copy(h_hbm.at[pl.ds(s * RPT, RPT)],
                    h_sh.at[pl.ds(s * RPT, RPT)])
    pltpu.sync_copy(zs_hbm.at[pl.ds(s * RPT, RPT)],
                    acc.at[pl.ds(s * RPT, RPT)])
    pltpu.sync_copy(src2_hbm.at[wid], srcf)
    pltpu.sync_copy(dst3_hbm.at[wid], dstv)
    plsc.subcore_barrier()

    def gat(j, buf, sem):
        pltpu.async_copy(h_sh.at[srcf.at[pl.ds(j * C, C)]], buf, sem)

    def wt(buf, sem):
        pltpu.make_async_copy(h_sh.at[srcf.at[pl.ds(0, C)]], buf, sem).wait()

    gat(0, rows_a, sem_a)

    def body(k, carry):
        a = 2 * k
        b = a + 1
        gat(b, rows_b, sem_b)
        wt(rows_a, sem_a)
        pltpu.sync_copy(rows_a, acc.at[dstv.at[a]], add=True)

        @pl.when(b + 1 < NCHUNK)
        def _():
            gat(b + 1, rows_a, sem_a)
        wt(rows_b, sem_b)
        pltpu.sync_copy(rows_b, acc.at[dstv.at[b]], add=True)
        return carry
    lax.fori_loop(0, NCHUNK // 2, body, 0)

    plsc.subcore_barrier()
    pltpu.sync_copy(acc.at[pl.ds(s * RPT, RPT)],
                    out_hbm.at[c, pl.ds(s * RPT, RPT)])


def _dense1(xp, W1, deg2):
    def body(x_ref, w_ref, d_ref, h_ref, dinv_ref):
        deg = d_ref[0][:, 0:1] + d_ref[1][:, 0:1] + 1.0
        dinv = lax.rsqrt(jnp.maximum(deg, 1.0))
        h = jnp.dot(x_ref[...], w_ref[...], preferred_element_type=jnp.float32)
        hp = h * dinv
        h_ref[0] = hp[:, :DH]
        h_ref[1] = hp[:, DH:]
        dinv_ref[...] = dinv
    return pl.pallas_call(
        body,
        out_shape=[jax.ShapeDtypeStruct((NC, NP, DH), jnp.float32),
                   jax.ShapeDtypeStruct((NP, 1), jnp.float32)],
    )(xp, W1, deg2)


def _dense_mid(S, hp2, dinv, b, g, be, W, DO):
    def body(S_ref, hp_ref, dinv_ref, b_ref, g_ref, be_ref, w_ref, o_ref):
        a = jnp.concatenate(
            [S_ref[0] + hp_ref[0], S_ref[1] + hp_ref[1]], axis=1)
        a = a * dinv_ref[...] + b_ref[...]
        aN = a[:N]
        m = jnp.mean(aN, axis=0, keepdims=True)
        d = aN - m
        v = jnp.mean(d * d, axis=0, keepdims=True)
        y = (a - m) * lax.rsqrt(v + 1e-5) * g_ref[...] + be_ref[...]
        y = jnp.maximum(y, 0.0)
        h = jnp.dot(y, w_ref[...], preferred_element_type=jnp.float32)
        hp = h * dinv_ref[...]
        if DO == 2 * DH:
            o_ref[0] = hp[:, :DH]
            o_ref[1] = hp[:, DH:]
        else:
            o_ref[...] = hp
    out_shape = (jax.ShapeDtypeStruct((NC, NP, DH), jnp.float32)
                 if DO == 2 * DH else
                 jax.ShapeDtypeStruct((NP, DO), jnp.float32))
    return pl.pallas_call(
        body,
        out_shape=out_shape,
    )(S, hp2, dinv, b, g, be, W)


def _dense_final(S, hp, dinv, b):
    def body(S_ref, hp_ref, dinv_ref, b_ref, o_ref):
        a = (S_ref[0] + S_ref[1] + hp_ref[...]) * dinv_ref[...] + b_ref[...]
        aN = a[:N, :D3]
        z = aN - jnp.max(aN, axis=1, keepdims=True)
        lse = jnp.log(jnp.sum(jnp.exp(z), axis=1, keepdims=True))
        o_ref[...] = z - lse
    return pl.pallas_call(
        body,
        out_shape=jax.ShapeDtypeStruct((N, D3), jnp.float32),
    )(S, hp, dinv, b)


def kernel(adj_t, x, W1, b1, g1, be1, W2, b2, g2, be2, W3, b3):
    f32 = jnp.float32
    pad = ((0, 0), (0, EPWP - EPW))
    src2 = jnp.pad(adj_t[0].astype(jnp.int32).reshape(NW, EPW), pad,
                   constant_values=NP - 1)
    dst3 = jnp.pad(adj_t[1].astype(jnp.int32).reshape(NW, EPW), pad,
                   constant_values=NP - 1).reshape(NW, NCHUNK, C)
    xp = jnp.pad(x.astype(f32), ((0, NP - N), (0, 0)))
    zs8 = jnp.zeros((NP, DW), f32)
    zs64 = jnp.zeros((NP, DH), f32)
    zs40 = jnp.zeros((NP, D3), f32)
    b1r = b1.reshape(1, 128)
    g1r = g1.reshape(1, 128)
    be1r = be1.reshape(1, 128)
    b2r = b2.reshape(1, 128)
    g2r = g2.reshape(1, 128)
    be2r = be2.reshape(1, 128)
    b3r = b3.reshape(1, D3)

    deg2 = _deg_kernel(dst3, zs8)
    h1p2, dinv = _dense1(xp, W1, deg2)
    S1 = _spmm_split(h1p2, src2, dst3, zs64)
    h2p2 = _dense_mid(S1, h1p2, dinv, b1r, g1r, be1r, W2, 128)
    S2 = _spmm_split(h2p2, src2, dst3, zs64)
    h3p = _dense_mid(S2, h2p2, dinv, b2r, g2r, be2r, W3, D3)
    S3 = _spmm40(h3p, src2, dst3, zs40)
    return _dense_final(S3, h3p, dinv, b3r)
